# Initial kernel scaffold; baseline (speedup 1.0000x reference)
import jax
import jax.numpy as jnp
from jax.experimental import pallas as pl


def kernel(x, edge_index, batch, W_emb, b_emb, W0, b0, W1, b1, W2, b2, W3, b3):
    raise NotImplementedError("write your pallas kernel here")


# speedup vs baseline: 6.9411x; 6.9411x over previous
import functools

import jax
import jax.numpy as jnp
from jax import lax
from jax.experimental import pallas as pl
from jax.experimental.pallas import tpu as pltpu
from jax.experimental.pallas import tpu_sc as plsc

N = 10000
NP = 10240
E = 640000
HP = 160
G = 64

NC = 2
NS = 16
NW = NC * NS
EPW = E // NW
K = 80
NCH = EPW // K
RPT = NP // NS
NZ = RPT // K


def _sc_agg_body(m_hbm, src_hbm, dst_hbm, out0, out1,
                 agg_sh, src_v, dst_v, rows_v, sem):
    c = lax.axis_index("c")
    s = lax.axis_index("s")
    wid = s * NC + c

    def zrow(r, carry):
        def zcol(q, carry2):
            rows_v[r, pl.ds(q * 16, 16)] = jnp.zeros((16,), jnp.float32)
            return carry2
        return lax.fori_loop(0, HP // 16, zcol, carry)
    lax.fori_loop(0, K, zrow, 0)

    row0 = s * RPT

    def zchunk(j, carry):
        pltpu.sync_copy(rows_v, agg_sh.at[pl.ds(row0 + j * K, K)])
        return carry
    lax.fori_loop(0, NZ, zchunk, 0)
    plsc.subcore_barrier()

    base = wid * EPW

    def body(i, carry):
        off = base + i * K
        pltpu.sync_copy(src_hbm.at[pl.ds(off, K)], src_v)
        pltpu.sync_copy(dst_hbm.at[pl.ds(off, K)], dst_v)
        pltpu.async_copy(m_hbm.at[src_v], rows_v, sem).wait()
        pltpu.sync_copy(rows_v, agg_sh.at[dst_v], add=True)
        return carry
    lax.fori_loop(0, NCH, body, 0)
    plsc.subcore_barrier()

    @pl.when(c == 0)
    def _():
        pltpu.sync_copy(agg_sh.at[pl.ds(row0, RPT)], out0.at[pl.ds(row0, RPT)])

    @pl.when(c == 1)
    def _():
        pltpu.sync_copy(agg_sh.at[pl.ds(row0, RPT)], out1.at[pl.ds(row0, RPT)])


_sc_agg = functools.partial(
    pl.kernel,
    out_type=[
        jax.ShapeDtypeStruct((NP, HP), jnp.float32),
        jax.ShapeDtypeStruct((NP, HP), jnp.float32),
    ],
    mesh=plsc.VectorSubcoreMesh(core_axis_name="c", subcore_axis_name="s"),
    compiler_params=pltpu.CompilerParams(use_tc_tiling_on_sc=False),
    scratch_types=[
        pltpu.VMEM_SHARED((NP, HP), jnp.float32),
        pltpu.VMEM((K,), jnp.int32),
        pltpu.VMEM((K,), jnp.int32),
        pltpu.VMEM((K, HP), jnp.float32),
        pltpu.SemaphoreType.DMA,
    ],
)(_sc_agg_body)


def _emb_body(x_ref, a_ref, b_ref, w_ref, o_ref):
    h = jnp.dot(x_ref[...], a_ref[...], preferred_element_type=jnp.float32)
    h = h + b_ref[...]
    o_ref[...] = jnp.dot(h, w_ref[...], preferred_element_type=jnp.float32)


def _layer_body(a0_ref, a1_ref, b_ref, w_ref, o_ref):
    h = jnp.maximum(a0_ref[...] + a1_ref[...] + b_ref[...], 0.0)
    o_ref[...] = jnp.dot(h, w_ref[...], preferred_element_type=jnp.float32)


def _pool_body(a0_ref, a1_ref, b_ref, batch_ref, o_ref):
    h = jnp.maximum(a0_ref[...] + a1_ref[...] + b_ref[...], 0.0)
    gids = lax.broadcasted_iota(jnp.int32, (G, NP), 0)
    mask = (batch_ref[...] == gids).astype(jnp.float32)
    sums = jnp.dot(mask, h, preferred_element_type=jnp.float32)
    counts = jnp.sum(mask, axis=1, keepdims=True)
    o_ref[...] = sums / jnp.maximum(counts, 1.0)


def _tc_emb(x, a, b, w):
    return pl.pallas_call(
        _emb_body,
        out_shape=jax.ShapeDtypeStruct((NP, HP), jnp.float32),
    )(x, a, b, w)


def _tc_layer(a0, a1, b, w):
    return pl.pallas_call(
        _layer_body,
        out_shape=jax.ShapeDtypeStruct((NP, HP), jnp.float32),
    )(a0, a1, b, w)


def _tc_pool(a0, a1, b, batch2d):
    return pl.pallas_call(
        _pool_body,
        out_shape=jax.ShapeDtypeStruct((G, HP), jnp.float32),
    )(a0, a1, b, batch2d)


def kernel(x, edge_index, batch, W_emb, b_emb, W0, b0, W1, b1, W2, b2, W3, b3):
    Hdim = W_emb.shape[0]
    pad = HP - Hdim

    a_emb = jnp.pad(W_emb.T, ((0, 0), (0, pad)))
    b_embp = jnp.pad(b_emb, (0, pad)).reshape(1, HP)
    ws = [jnp.pad(W.T, ((0, pad), (0, pad))) for W in (W0, W1, W2, W3)]
    bs = [jnp.pad(b, (0, pad)).reshape(1, HP) for b in (b0, b1, b2, b3)]

    src = edge_index[0]
    dst = edge_index[1]
    x_p = jnp.pad(x, ((0, NP - N), (0, 0)))
    batch2d = jnp.pad(batch, (0, NP - N), constant_values=G).reshape(1, NP)

    m = _tc_emb(x_p, a_emb, b_embp, ws[0])
    agg0, agg1 = _sc_agg(m, src, dst)
    for i in range(1, 4):
        m = _tc_layer(agg0, agg1, bs[i - 1], ws[i])
        agg0, agg1 = _sc_agg(m, src, dst)
    out = _tc_pool(agg0, agg1, bs[3], batch2d)
    return out[:, :Hdim]


# speedup vs baseline: 13.7116x; 1.9754x over previous
import functools

import jax
import jax.numpy as jnp
from jax import lax
from jax.experimental import pallas as pl
from jax.experimental.pallas import tpu as pltpu
from jax.experimental.pallas import tpu_sc as plsc

N = 10000
NP = 10240
E = 640000
HP = 160
G = 64

NC = 2
NS = 16
NW = NC * NS
EPW = E // NW
K = 80
NCH = EPW // K
CB = 10
NB = NCH // CB
RPT = NP // NS
NZ = RPT // K


def _sc_agg_body(m_hbm, src_hbm, dst_hbm, out0, out1,
                 agg_sh, src_blk, dst_blk, rows_v, sem0, sem1):
    c = lax.axis_index("c")
    s = lax.axis_index("s")
    wid = s * NC + c

    zrows = rows_v.at[0]

    def zrow(r, carry):
        def zcol(q, carry2):
            zrows[r, pl.ds(q * 16, 16)] = jnp.zeros((16,), jnp.float32)
            return carry2
        return lax.fori_loop(0, HP // 16, zcol, carry)
    lax.fori_loop(0, K, zrow, 0)

    row0 = s * RPT

    def zchunk(j, carry):
        pltpu.sync_copy(zrows, agg_sh.at[pl.ds(row0 + j * K, K)])
        return carry
    lax.fori_loop(0, NZ, zchunk, 0)
    plsc.subcore_barrier()

    base_row = wid * NCH
    sems = (sem0, sem1)

    def block(b, carry):
        r0 = base_row + b * CB
        pltpu.sync_copy(src_hbm.at[pl.ds(r0, CB)], src_blk)
        pltpu.sync_copy(dst_hbm.at[pl.ds(r0, CB)], dst_blk)
        pltpu.async_copy(m_hbm.at[src_blk.at[0]], rows_v.at[0], sems[0])
        for j in range(CB):
            jj = j % 2
            if j + 1 < CB:
                pltpu.async_copy(m_hbm.at[src_blk.at[j + 1]],
                                 rows_v.at[1 - jj], sems[1 - jj])
            pltpu.make_async_copy(m_hbm.at[src_blk.at[j]],
                                  rows_v.at[jj], sems[jj]).wait()
            pltpu.sync_copy(rows_v.at[jj], agg_sh.at[dst_blk.at[j]], add=True)
        return carry
    lax.fori_loop(0, NB, block, 0)
    plsc.subcore_barrier()

    @pl.when(c == 0)
    def _():
        pltpu.sync_copy(agg_sh.at[pl.ds(row0, RPT)], out0.at[pl.ds(row0, RPT)])

    @pl.when(c == 1)
    def _():
        pltpu.sync_copy(agg_sh.at[pl.ds(row0, RPT)], out1.at[pl.ds(row0, RPT)])


_sc_agg = functools.partial(
    pl.kernel,
    out_type=[
        jax.ShapeDtypeStruct((NP, HP), jnp.float32),
        jax.ShapeDtypeStruct((NP, HP), jnp.float32),
    ],
    mesh=plsc.VectorSubcoreMesh(core_axis_name="c", subcore_axis_name="s"),
    compiler_params=pltpu.CompilerParams(use_tc_tiling_on_sc=False),
    scratch_types=[
        pltpu.VMEM_SHARED((NP, HP), jnp.float32),
        pltpu.VMEM((CB, K), jnp.int32),
        pltpu.VMEM((CB, K), jnp.int32),
        pltpu.VMEM((2, K, HP), jnp.float32),
        pltpu.SemaphoreType.DMA,
        pltpu.SemaphoreType.DMA,
    ],
)(_sc_agg_body)


def _emb_body(x_ref, a_ref, b_ref, w_ref, o_ref):
    h = jnp.dot(x_ref[...], a_ref[...], preferred_element_type=jnp.float32)
    h = h + b_ref[...]
    o_ref[...] = jnp.dot(h, w_ref[...], preferred_element_type=jnp.float32)


def _layer_body(a0_ref, a1_ref, b_ref, w_ref, o_ref):
    h = jnp.maximum(a0_ref[...] + a1_ref[...] + b_ref[...], 0.0)
    o_ref[...] = jnp.dot(h, w_ref[...], preferred_element_type=jnp.float32)


def _pool_body(a0_ref, a1_ref, b_ref, batch_ref, o_ref):
    h = jnp.maximum(a0_ref[...] + a1_ref[...] + b_ref[...], 0.0)
    gids = lax.broadcasted_iota(jnp.int32, (G, NP), 0)
    mask = (batch_ref[...] == gids).astype(jnp.float32)
    sums = jnp.dot(mask, h, preferred_element_type=jnp.float32)
    counts = jnp.sum(mask, axis=1, keepdims=True)
    o_ref[...] = sums / jnp.maximum(counts, 1.0)


def _tc_emb(x, a, b, w):
    return pl.pallas_call(
        _emb_body,
        out_shape=jax.ShapeDtypeStruct((NP, HP), jnp.float32),
    )(x, a, b, w)


def _tc_layer(a0, a1, b, w):
    return pl.pallas_call(
        _layer_body,
        out_shape=jax.ShapeDtypeStruct((NP, HP), jnp.float32),
    )(a0, a1, b, w)


def _tc_pool(a0, a1, b, batch2d):
    return pl.pallas_call(
        _pool_body,
        out_shape=jax.ShapeDtypeStruct((G, HP), jnp.float32),
    )(a0, a1, b, batch2d)


def kernel(x, edge_index, batch, W_emb, b_emb, W0, b0, W1, b1, W2, b2, W3, b3):
    Hdim = W_emb.shape[0]
    pad = HP - Hdim

    a_emb = jnp.pad(W_emb.T, ((0, 0), (0, pad)))
    b_embp = jnp.pad(b_emb, (0, pad)).reshape(1, HP)
    ws = [jnp.pad(W.T, ((0, pad), (0, pad))) for W in (W0, W1, W2, W3)]
    bs = [jnp.pad(b, (0, pad)).reshape(1, HP) for b in (b0, b1, b2, b3)]

    src = edge_index[0].reshape(E // K, K)
    dst = edge_index[1].reshape(E // K, K)
    x_p = jnp.pad(x, ((0, NP - N), (0, 0)))
    batch2d = jnp.pad(batch, (0, NP - N), constant_values=G).reshape(1, NP)

    m = _tc_emb(x_p, a_emb, b_embp, ws[0])
    agg0, agg1 = _sc_agg(m, src, dst)
    for i in range(1, 4):
        m = _tc_layer(agg0, agg1, bs[i - 1], ws[i])
        agg0, agg1 = _sc_agg(m, src, dst)
    out = _tc_pool(agg0, agg1, bs[3], batch2d)
    return out[:, :Hdim]


# speedup vs baseline: 15.0117x; 1.0948x over previous
import functools

import jax
import jax.numpy as jnp
from jax import lax
from jax.experimental import pallas as pl
from jax.experimental.pallas import tpu as pltpu
from jax.experimental.pallas import tpu_sc as plsc

N = 10000
NP = 10112
E = 640000
HP = 160
G = 64

NC = 2
NS = 16
NW = NC * NS
EPW = E // NW
K = 80
NCH = EPW // K
CB = 25
NB = NCH // CB
RPT = NP // NS
NZ = RPT // K


def _sc_agg_body(m_hbm, src_hbm, dst_hbm, out0, out1,
                 agg_sh, src_blk, dst_blk, rows_v, sem0, sem1):
    c = lax.axis_index("c")
    s = lax.axis_index("s")
    wid = s * NC + c

    zrows = rows_v.at[0]

    def zrow(r, carry):
        def zcol(q, carry2):
            zrows[r, pl.ds(q * 16, 16)] = jnp.zeros((16,), jnp.float32)
            return carry2
        return lax.fori_loop(0, HP // 16, zcol, carry)
    lax.fori_loop(0, K, zrow, 0)

    row0 = s * RPT

    def zchunk(j, carry):
        pltpu.sync_copy(zrows, agg_sh.at[pl.ds(row0 + j * K, K)])
        return carry
    lax.fori_loop(0, NZ, zchunk, 0)
    zt = RPT - NZ * K
    if zt:
        pltpu.sync_copy(rows_v.at[0, pl.ds(0, zt)],
                        agg_sh.at[pl.ds(row0 + NZ * K, zt)])
    plsc.subcore_barrier()

    base_row = wid * NCH
    sems = (sem0, sem1)

    def block(b, carry):
        r0 = base_row + b * CB
        pltpu.sync_copy(src_hbm.at[pl.ds(r0, CB)], src_blk)
        pltpu.sync_copy(dst_hbm.at[pl.ds(r0, CB)], dst_blk)
        pltpu.async_copy(m_hbm.at[src_blk.at[0]], rows_v.at[0], sems[0])
        for j in range(CB):
            jj = j % 2
            if j + 1 < CB:
                pltpu.async_copy(m_hbm.at[src_blk.at[j + 1]],
                                 rows_v.at[1 - jj], sems[1 - jj])
            pltpu.make_async_copy(m_hbm.at[src_blk.at[j]],
                                  rows_v.at[jj], sems[jj]).wait()
            pltpu.sync_copy(rows_v.at[jj], agg_sh.at[dst_blk.at[j]], add=True)
        return carry
    lax.fori_loop(0, NB, block, 0)
    plsc.subcore_barrier()

    @pl.when(c == 0)
    def _():
        pltpu.sync_copy(agg_sh.at[pl.ds(row0, RPT)], out0.at[pl.ds(row0, RPT)])

    @pl.when(c == 1)
    def _():
        pltpu.sync_copy(agg_sh.at[pl.ds(row0, RPT)], out1.at[pl.ds(row0, RPT)])


_sc_agg = functools.partial(
    pl.kernel,
    out_type=[
        jax.ShapeDtypeStruct((NP, HP), jnp.float32),
        jax.ShapeDtypeStruct((NP, HP), jnp.float32),
    ],
    mesh=plsc.VectorSubcoreMesh(core_axis_name="c", subcore_axis_name="s"),
    compiler_params=pltpu.CompilerParams(use_tc_tiling_on_sc=False),
    scratch_types=[
        pltpu.VMEM_SHARED((NP, HP), jnp.float32),
        pltpu.VMEM((CB, K), jnp.int32),
        pltpu.VMEM((CB, K), jnp.int32),
        pltpu.VMEM((2, K, HP), jnp.float32),
        pltpu.SemaphoreType.DMA,
        pltpu.SemaphoreType.DMA,
    ],
)(_sc_agg_body)


def _emb_body(x_ref, a_ref, b_ref, w_ref, o_ref):
    h = jnp.dot(x_ref[...], a_ref[...], preferred_element_type=jnp.float32)
    h = h + b_ref[...]
    o_ref[...] = jnp.dot(h, w_ref[...], preferred_element_type=jnp.float32)


def _layer_body(a0_ref, a1_ref, b_ref, w_ref, o_ref):
    h = jnp.maximum(a0_ref[...] + a1_ref[...] + b_ref[...], 0.0)
    o_ref[...] = jnp.dot(h, w_ref[...], preferred_element_type=jnp.float32)


def _pool_body(a0_ref, a1_ref, b_ref, batch_ref, o_ref):
    h = jnp.maximum(a0_ref[...] + a1_ref[...] + b_ref[...], 0.0)
    gids = lax.broadcasted_iota(jnp.int32, (G, NP), 0)
    mask = (batch_ref[...] == gids).astype(jnp.float32)
    sums = jnp.dot(mask, h, preferred_element_type=jnp.float32)
    counts = jnp.sum(mask, axis=1, keepdims=True)
    o_ref[...] = sums / jnp.maximum(counts, 1.0)


def _tc_emb(x, a, b, w):
    return pl.pallas_call(
        _emb_body,
        out_shape=jax.ShapeDtypeStruct((NP, HP), jnp.float32),
    )(x, a, b, w)


def _tc_layer(a0, a1, b, w):
    return pl.pallas_call(
        _layer_body,
        out_shape=jax.ShapeDtypeStruct((NP, HP), jnp.float32),
    )(a0, a1, b, w)


def _tc_pool(a0, a1, b, batch2d):
    return pl.pallas_call(
        _pool_body,
        out_shape=jax.ShapeDtypeStruct((G, HP), jnp.float32),
    )(a0, a1, b, batch2d)


def kernel(x, edge_index, batch, W_emb, b_emb, W0, b0, W1, b1, W2, b2, W3, b3):
    Hdim = W_emb.shape[0]
    pad = HP - Hdim

    a_emb = jnp.pad(W_emb.T, ((0, 0), (0, pad)))
    b_embp = jnp.pad(b_emb, (0, pad)).reshape(1, HP)
    ws = [jnp.pad(W.T, ((0, pad), (0, pad))) for W in (W0, W1, W2, W3)]
    bs = [jnp.pad(b, (0, pad)).reshape(1, HP) for b in (b0, b1, b2, b3)]

    src = edge_index[0].reshape(E // K, K)
    dst = edge_index[1].reshape(E // K, K)
    x_p = jnp.pad(x, ((0, NP - N), (0, 0)))
    batch2d = jnp.pad(batch, (0, NP - N), constant_values=G).reshape(1, NP)

    m = _tc_emb(x_p, a_emb, b_embp, ws[0])
    agg0, agg1 = _sc_agg(m, src, dst)
    for i in range(1, 4):
        m = _tc_layer(agg0, agg1, bs[i - 1], ws[i])
        agg0, agg1 = _sc_agg(m, src, dst)
    out = _tc_pool(agg0, agg1, bs[3], batch2d)
    return out[:, :Hdim]
